# Initial kernel scaffold; baseline (speedup 1.0000x reference)
import jax
import jax.numpy as jnp
from jax.experimental import pallas as pl


def kernel(x, edge_index, W1, b1, W2, b2):
    raise NotImplementedError("write your pallas kernel here")


# speedup vs baseline: 33.4140x; 33.4140x over previous
import functools

import jax
import jax.numpy as jnp
from jax import lax
from jax.experimental import pallas as pl
from jax.experimental.pallas import tpu as pltpu
from jax.experimental.pallas import tpu_sc as plsc

N = 10000
NP = 10240
D_IN = 128
H = 16
C = 40
E = 320000

NC = 2
NS = 16
NW = NC * NS
STRIPE = NP // NS
CHUNK = 128
CPT = 80
NBUF = 4
E_PAD = NW * CPT * CHUNK

ROWS_BLK = 1024
GRID = NP // ROWS_BLK


_MESH = plsc.VectorSubcoreMesh(core_axis_name="c", subcore_axis_name="s")


def _deg_body(dst_hbm, zeros1_hbm, ones_hbm, out_hbm, idx_d, ones_v, acc1):
    c = lax.axis_index("c")
    s = lax.axis_index("s")
    pltpu.sync_copy(zeros1_hbm.at[pl.ds(s * STRIPE, STRIPE)],
                    acc1.at[pl.ds(s * STRIPE, STRIPE)])
    w = c * NS + s
    pltpu.sync_copy(dst_hbm.at[w], idx_d)
    pltpu.sync_copy(ones_hbm, ones_v)
    plsc.subcore_barrier()

    def body(j, carry):
        pltpu.sync_copy(ones_v, acc1.at[idx_d.at[j]], add=True)
        return carry

    lax.fori_loop(0, CPT, body, 0)
    plsc.subcore_barrier()
    pltpu.sync_copy(acc1.at[pl.ds(s * STRIPE, STRIPE)],
                    out_hbm.at[c, pl.ds(s * STRIPE, STRIPE)])


_SC_PARAMS = pltpu.CompilerParams(use_tc_tiling_on_sc=False)

_deg_call = functools.partial(
    pl.kernel,
    mesh=_MESH,
    compiler_params=_SC_PARAMS,
    out_type=jax.ShapeDtypeStruct((NC, NP), jnp.float32),
    scratch_types=[
        pltpu.VMEM((CPT, CHUNK), jnp.int32),
        pltpu.VMEM((CHUNK,), jnp.float32),
        pltpu.VMEM_SHARED((NP,), jnp.float32),
    ],
)(_deg_body)


def _edge_body(y_hbm, src_hbm, dst_hbm, zeros_hbm, out_hbm,
               idx_s, idx_d, rows, acc, sem):
    c = lax.axis_index("c")
    s = lax.axis_index("s")
    pltpu.sync_copy(zeros_hbm.at[pl.ds(s * STRIPE, STRIPE)],
                    acc.at[pl.ds(s * STRIPE, STRIPE)])
    w = c * NS + s
    pltpu.sync_copy(src_hbm.at[w], idx_s)
    pltpu.sync_copy(dst_hbm.at[w], idx_d)
    plsc.subcore_barrier()

    def body(g, carry):
        base = g * NBUF
        cps = [pltpu.async_copy(y_hbm.at[idx_s.at[base + b]], rows.at[b], sem)
               for b in range(NBUF)]
        for cp in cps:
            cp.wait()
        for b in range(NBUF):
            pltpu.sync_copy(rows.at[b], acc.at[idx_d.at[base + b]], add=True)
        return carry

    lax.fori_loop(0, CPT // NBUF, body, 0)
    plsc.subcore_barrier()
    pltpu.sync_copy(acc.at[pl.ds(s * STRIPE, STRIPE)],
                    out_hbm.at[c, pl.ds(s * STRIPE, STRIPE)])


_edge_call = functools.partial(
    pl.kernel,
    mesh=_MESH,
    compiler_params=_SC_PARAMS,
    out_type=jax.ShapeDtypeStruct((NC, NP, H), jnp.float32),
    scratch_types=[
        pltpu.VMEM((CPT, CHUNK), jnp.int32),
        pltpu.VMEM((CPT, CHUNK), jnp.int32),
        pltpu.VMEM((NBUF, CHUNK, H), jnp.float32),
        pltpu.VMEM_SHARED((NP, H), jnp.float32),
        pltpu.SemaphoreType.DMA,
    ],
)(_edge_body)


def _tc1_body(x_ref, w1_ref, degt_ref, y1_ref, dis_ref):
    xw = jnp.dot(x_ref[...], w1_ref[...], preferred_element_type=jnp.float32)
    deg = jnp.sum(degt_ref[...], axis=1, keepdims=True) + 1.0
    dis = lax.rsqrt(deg)
    y1_ref[...] = xw * dis
    dis_ref[...] = dis


_tc1 = pl.pallas_call(
    _tc1_body,
    grid=(GRID,),
    in_specs=[
        pl.BlockSpec((ROWS_BLK, D_IN), lambda i: (i, 0)),
        pl.BlockSpec((D_IN, H), lambda i: (0, 0)),
        pl.BlockSpec((ROWS_BLK, NC), lambda i: (i, 0)),
    ],
    out_specs=[
        pl.BlockSpec((ROWS_BLK, H), lambda i: (i, 0)),
        pl.BlockSpec((ROWS_BLK, 1), lambda i: (i, 0)),
    ],
    out_shape=[
        jax.ShapeDtypeStruct((NP, H), jnp.float32),
        jax.ShapeDtypeStruct((NP, 1), jnp.float32),
    ],
)


def _tc2_body(aggp_ref, y1_ref, dis_ref, b1_ref, u_ref):
    agg = aggp_ref[0] + aggp_ref[1] + y1_ref[...]
    h = jnp.maximum(agg * dis_ref[...] + b1_ref[...], 0.0)
    u_ref[...] = h * dis_ref[...]


_tc2 = pl.pallas_call(
    _tc2_body,
    grid=(GRID,),
    in_specs=[
        pl.BlockSpec((NC, ROWS_BLK, H), lambda i: (0, i, 0)),
        pl.BlockSpec((ROWS_BLK, H), lambda i: (i, 0)),
        pl.BlockSpec((ROWS_BLK, 1), lambda i: (i, 0)),
        pl.BlockSpec((1, H), lambda i: (0, 0)),
    ],
    out_specs=pl.BlockSpec((ROWS_BLK, H), lambda i: (i, 0)),
    out_shape=jax.ShapeDtypeStruct((NP, H), jnp.float32),
)


def _tc3_body(aggp_ref, u_ref, dis_ref, w2_ref, b2_ref, out_ref):
    g = (aggp_ref[0] + aggp_ref[1] + u_ref[...]) * dis_ref[...]
    z = jnp.dot(g, w2_ref[...], preferred_element_type=jnp.float32) + b2_ref[...]
    m = jnp.max(z, axis=1, keepdims=True)
    lse = jnp.log(jnp.sum(jnp.exp(z - m), axis=1, keepdims=True))
    out_ref[...] = z - m - lse


_tc3 = pl.pallas_call(
    _tc3_body,
    grid=(GRID,),
    in_specs=[
        pl.BlockSpec((NC, ROWS_BLK, H), lambda i: (0, i, 0)),
        pl.BlockSpec((ROWS_BLK, H), lambda i: (i, 0)),
        pl.BlockSpec((ROWS_BLK, 1), lambda i: (i, 0)),
        pl.BlockSpec((H, C), lambda i: (0, 0)),
        pl.BlockSpec((1, C), lambda i: (0, 0)),
    ],
    out_specs=pl.BlockSpec((ROWS_BLK, C), lambda i: (i, 0)),
    out_shape=jax.ShapeDtypeStruct((NP, C), jnp.float32),
)


def kernel(x, edge_index, W1, b1, W2, b2):
    src = edge_index[0].astype(jnp.int32)
    dst = edge_index[1].astype(jnp.int32)
    pad = E_PAD - E
    fill = jnp.full((pad,), N, jnp.int32)
    srcp = jnp.concatenate([src, fill]).reshape(NW, CPT, CHUNK)
    dstp = jnp.concatenate([dst, fill]).reshape(NW, CPT, CHUNK)

    x_pad = jnp.pad(x, ((0, NP - N), (0, 0)))
    zeros1 = jnp.zeros((NP,), jnp.float32)
    zeros2 = jnp.zeros((NP, H), jnp.float32)
    ones = jnp.ones((CHUNK,), jnp.float32)

    degp = _deg_call(dstp, zeros1, ones)
    degt = degp.T
    y1, dis = _tc1(x_pad, W1, degt)
    agg1 = _edge_call(y1, srcp, dstp, zeros2)
    u = _tc2(agg1, y1, dis, b1.reshape(1, H))
    agg2 = _edge_call(u, srcp, dstp, zeros2)
    outp = _tc3(agg2, u, dis, W2, b2.reshape(1, C))
    return outp[:N]
